# baseline (speedup 1.0000x reference)
import functools

import jax
import jax.numpy as jnp
from jax import lax
from jax.experimental import pallas as pl
from jax.experimental.pallas import tpu as pltpu
from jax.experimental.pallas import tpu_sc as plsc

N = 10000
E = 160000
NC = 2
NS = 16
NW = NC * NS
EPW = E // NW
BATCH = 128
NBATCH = 40
EPW_PAD = NBATCH * BATCH
NP = 10240
RPT = NP // NS


def _make_segsum(C, W):
  mesh = plsc.VectorSubcoreMesh(
      core_axis_name="c", subcore_axis_name="s", num_cores=NC,
      num_subcores=NS)
  npair = NBATCH // 2

  @functools.partial(
      pl.kernel,
      out_type=jax.ShapeDtypeStruct((NC * C, NP, W), jnp.float32),
      mesh=mesh,
      scratch_types=[
          pltpu.VMEM_SHARED((NP, W), jnp.float32),
          pltpu.VMEM((NBATCH, BATCH), jnp.int32),
          pltpu.VMEM((NBATCH, BATCH), jnp.int32),
          pltpu.VMEM((BATCH, W), jnp.float32),
          pltpu.VMEM((BATCH, W), jnp.float32),
          pltpu.SemaphoreType.DMA,
          pltpu.SemaphoreType.DMA,
      ],
  )
  def segsum(h2d, src, dst, zeros, out, agg, srcv, dstv, rows0, rows1,
             sem0, sem1):
    cid = lax.axis_index("c")
    sid = lax.axis_index("s")
    wid = cid * NS + sid

    pltpu.sync_copy(dst.at[wid], dstv)

    for c in range(C):
      pltpu.sync_copy(src.at[c * NW + wid], srcv)
      pltpu.sync_copy(zeros, agg.at[pl.ds(sid * RPT, RPT)])
      plsc.subcore_barrier()

      pltpu.make_async_copy(h2d.at[srcv.at[0]], rows0, sem0).start()

      def pair(jj, carry):
        j0 = jj * 2
        j1 = j0 + 1
        pltpu.make_async_copy(h2d.at[srcv.at[j0]], rows0, sem0).wait()
        pltpu.make_async_copy(h2d.at[srcv.at[j1]], rows1, sem1).start()
        pltpu.sync_copy(rows0, agg.at[dstv.at[j0]], add=True)
        pltpu.make_async_copy(h2d.at[srcv.at[j1]], rows1, sem1).wait()

        @pl.when(jj + 1 < npair)
        def _():
          pltpu.make_async_copy(h2d.at[srcv.at[j0 + 2]], rows0, sem0).start()

        pltpu.sync_copy(rows1, agg.at[dstv.at[j1]], add=True)
        return carry

      lax.fori_loop(0, npair, pair, 0)

      plsc.subcore_barrier()
      pltpu.sync_copy(
          agg.at[pl.ds(sid * RPT, RPT)],
          out.at[cid * C + c, pl.ds(sid * RPT, RPT)])

  return segsum


NBH = 40
EPT = E // NS


def _make_segsum_cc():
  mesh = plsc.VectorSubcoreMesh(
      core_axis_name="c", subcore_axis_name="s", num_cores=NC,
      num_subcores=NS)
  npair = NBH // 2

  @functools.partial(
      pl.kernel,
      out_type=jax.ShapeDtypeStruct((4, NP, 128), jnp.float32),
      mesh=mesh,
      scratch_types=[
          pltpu.VMEM_SHARED((NP, 128), jnp.float32),
          pltpu.VMEM((NBH, BATCH), jnp.int32),
          pltpu.VMEM((NBH, BATCH), jnp.int32),
          pltpu.VMEM((BATCH, 128), jnp.float32),
          pltpu.VMEM((BATCH, 128), jnp.float32),
          pltpu.SemaphoreType.DMA,
          pltpu.SemaphoreType.DMA,
          pltpu.SemaphoreType.DMA,
          pltpu.SemaphoreType.DMA,
      ],
  )
  def segsum(h2d, src, dst, zeros, out, agg, srcv, dstv, rows0, rows1,
             sem0, sem1, sem2, sem3):
    cid = lax.axis_index("c")
    sid = lax.axis_index("s")

    def chunk(k, carry):
      c = cid * 2 + k
      pltpu.sync_copy(zeros, agg.at[pl.ds(sid * RPT, RPT)])
      plsc.subcore_barrier()

      def half(m, carry2):
        pltpu.sync_copy(src.at[(c * 2 + m) * NS + sid], srcv)
        pltpu.sync_copy(dst.at[m * NS + sid], dstv)
        pltpu.make_async_copy(h2d.at[srcv.at[0]], rows0, sem0).start()
        pltpu.make_async_copy(h2d.at[srcv.at[1]], rows1, sem1).start()

        def pair(jj, carry3):
          j0 = jj * 2
          j1 = j0 + 1
          pltpu.make_async_copy(h2d.at[srcv.at[j0]], rows0, sem0).wait()
          da = pltpu.async_copy(rows0, agg.at[dstv.at[j0]], sem2, add=True)
          pltpu.make_async_copy(h2d.at[srcv.at[j1]], rows1, sem1).wait()
          db = pltpu.async_copy(rows1, agg.at[dstv.at[j1]], sem3, add=True)
          da.wait()

          @pl.when(jj + 1 < npair)
          def _():
            pltpu.make_async_copy(h2d.at[srcv.at[j0 + 2]], rows0,
                                  sem0).start()

          db.wait()

          @pl.when(jj + 1 < npair)
          def _():
            pltpu.make_async_copy(h2d.at[srcv.at[j1 + 2]], rows1,
                                  sem1).start()

          return carry3

        lax.fori_loop(0, npair, pair, 0)
        return carry2

      lax.fori_loop(0, 2, half, 0)

      plsc.subcore_barrier()
      pltpu.sync_copy(
          agg.at[pl.ds(sid * RPT, RPT)],
          out.at[c, pl.ds(sid * RPT, RPT)])
      return carry

    lax.fori_loop(0, 2, chunk, 0)

  return segsum


_RB = 2000
_TRB = 2000


def _mm(xs, w, c_out):
  ks = [x.shape[1] for x in xs]
  offs = [sum(ks[:i]) for i in range(len(ks))]

  def body(*refs):
    xrefs, w_ref, o_ref = refs[:-2], refs[-2], refs[-1]
    acc = None
    for x_ref, k, off in zip(xrefs, ks, offs):
      part = lax.dot_general(
          x_ref[...], w_ref[off:off + k, :], (((1,), (0,)), ((), ())),
          precision=lax.Precision.DEFAULT,
          preferred_element_type=jnp.float32)
      acc = part if acc is None else acc + part
    for co in range(c_out):
      o_ref[co] = acc[:, co * 128:(co + 1) * 128]

  return pl.pallas_call(
      body,
      grid=(N // _RB,),
      in_specs=[pl.BlockSpec((_RB, k), lambda r: (r, 0)) for k in ks]
      + [pl.BlockSpec(w.shape, lambda r: (0, 0))],
      out_specs=pl.BlockSpec((c_out, _RB, 128), lambda r: (0, r, 0)),
      out_shape=jax.ShapeDtypeStruct((c_out, N, 128), jnp.float32),
  )(*xs, w)


def _tanh_sum_body(a_ref, b_ref, o_ref):
  o_ref[...] = jnp.tanh(a_ref[0] + b_ref[0]).astype(o_ref.dtype)


def _tanh_sum(p, C, dtype=jnp.float32):
  return pl.pallas_call(
      _tanh_sum_body,
      grid=(C, N // _TRB),
      in_specs=[
          pl.BlockSpec((1, _TRB, 128), lambda c, r: (c, r, 0)),
          pl.BlockSpec((1, _TRB, 128), lambda c, r: (C + c, r, 0)),
      ],
      out_specs=pl.BlockSpec((_TRB, 128), lambda c, r: (r, c)),
      out_shape=jax.ShapeDtypeStruct((N, C * 128), dtype),
  )(p, p)


def _tanh_mm(p, xs, w, c_out):
  ks = [x.shape[1] for x in xs]
  offs = [sum(ks[:i]) for i in range(len(ks))]
  base = sum(ks)

  def body(*refs):
    p_ref = refs[0]
    xrefs = refs[1:1 + len(xs)]
    w_ref = refs[1 + len(xs)]
    x_out, h_out = refs[2 + len(xs):]
    xnew = jnp.concatenate([jnp.tanh(p_ref[c]) for c in range(4)],
                           axis=1).astype(jnp.bfloat16)
    x_out[...] = xnew
    acc = lax.dot_general(
        xnew, w_ref[base:base + 512, :], (((1,), (0,)), ((), ())),
        precision=lax.Precision.DEFAULT,
        preferred_element_type=jnp.float32)
    for x_ref, k, off in zip(xrefs, ks, offs):
      acc += lax.dot_general(
          x_ref[...], w_ref[off:off + k, :], (((1,), (0,)), ((), ())),
          precision=lax.Precision.DEFAULT,
          preferred_element_type=jnp.float32)
    for co in range(c_out):
      h_out[co] = acc[:, co * 128:(co + 1) * 128]

  return pl.pallas_call(
      body,
      grid=(N // _RB,),
      in_specs=[pl.BlockSpec((4, _RB, 128), lambda r: (0, r, 0))]
      + [pl.BlockSpec((_RB, k), lambda r: (r, 0)) for k in ks]
      + [pl.BlockSpec(w.shape, lambda r: (0, 0))],
      out_specs=[
          pl.BlockSpec((_RB, 512), lambda r: (r, 0)),
          pl.BlockSpec((c_out, _RB, 128), lambda r: (0, r, 0)),
      ],
      out_shape=[
          jax.ShapeDtypeStruct((N, 512), jnp.bfloat16),
          jax.ShapeDtypeStruct((c_out, N, 128), jnp.float32),
      ],
  )(p, *xs, w)


def kernel(features, edge_index, W0, W1, W2, W3):
  src = edge_index[0].astype(jnp.int32)
  dst = edge_index[1].astype(jnp.int32)

  npad = EPW_PAD - EPW
  pad_src = jnp.broadcast_to((jnp.arange(npad, dtype=jnp.int32) * 83) % N,
                             (NW, npad))
  pad_dst = jnp.broadcast_to(
      N + (jnp.arange(npad, dtype=jnp.int32) % (NP - N)), (NW, npad))
  src_w = jnp.concatenate([src.reshape(NW, EPW), pad_src], axis=1)
  dst_w = jnp.concatenate([dst.reshape(NW, EPW), pad_dst], axis=1)
  dst_t = dst_w.reshape(NW, NBATCH, BATCH)
  src1 = src_w.reshape(NW, NBATCH, BATCH)

  nptc = 2 * NBH * BATCH - EPT
  pad_src_t = jnp.broadcast_to((jnp.arange(nptc, dtype=jnp.int32) * 83) % N,
                               (NS, nptc))
  pad_dst_t = jnp.broadcast_to(
      N + (jnp.arange(nptc, dtype=jnp.int32) % (NP - N)), (NS, nptc))
  src_tile = jnp.concatenate([src.reshape(NS, EPT), pad_src_t], axis=1)
  dst_tile = jnp.concatenate([dst.reshape(NS, EPT), pad_dst_t], axis=1)
  dst_cc = dst_tile.reshape(NS, 2, NBH, BATCH).transpose(1, 0, 2, 3)
  dst_cc = dst_cc.reshape(2 * NS, NBH, BATCH)
  src_cc = (jnp.arange(4, dtype=jnp.int32)[:, None, None] * N
            + src_tile[None]).reshape(4, NS, 2, NBH, BATCH)
  src_cc = src_cc.transpose(0, 2, 1, 3, 4).reshape(8 * NS, NBH, BATCH)

  zeros128 = jnp.zeros((RPT, 128), jnp.float32)
  seg_cc = _make_segsum_cc()
  seg1 = _make_segsum(1, 128)

  featb = features.astype(jnp.bfloat16)
  W0b = W0.astype(jnp.bfloat16)
  W1b = W1.astype(jnp.bfloat16)
  W2b = W2.astype(jnp.bfloat16)

  h0 = _mm([featb], W0b, 4)
  p0 = seg_cc(h0.reshape(4 * N, 128), src_cc, dst_cc, zeros128)

  x0, h1 = _tanh_mm(p0, [], W1b, 4)
  p1 = seg_cc(h1.reshape(4 * N, 128), src_cc, dst_cc, zeros128)

  x1, h2 = _tanh_mm(p1, [x0], W2b, 4)
  p2 = seg_cc(h2.reshape(4 * N, 128), src_cc, dst_cc, zeros128)

  W3p = jnp.pad(W3, ((0, 0), (0, 64))).astype(jnp.bfloat16)
  _, h3 = _tanh_mm(p2, [x0, x1], W3p, 1)
  p3 = seg1(h3.reshape(N, 128), src1, dst_t, zeros128)
  out = _tanh_sum(p3, 1)
  return out[:, :64]


# speedup vs baseline: 1.0957x; 1.0957x over previous
import functools

import jax
import jax.numpy as jnp
from jax import lax
from jax.experimental import pallas as pl
from jax.experimental.pallas import tpu as pltpu
from jax.experimental.pallas import tpu_sc as plsc

N = 10000
E = 160000
NC = 2
NS = 16
NW = NC * NS
EPW = E // NW
BATCH = 128
NBATCH = 40
EPW_PAD = NBATCH * BATCH
NP = 10240
RPT = NP // NS


def _make_segsum(C, W):
  mesh = plsc.VectorSubcoreMesh(
      core_axis_name="c", subcore_axis_name="s", num_cores=NC,
      num_subcores=NS)
  npair = NBATCH // 2

  @functools.partial(
      pl.kernel,
      out_type=jax.ShapeDtypeStruct((NC * C, NP, W), jnp.float32),
      mesh=mesh,
      scratch_types=[
          pltpu.VMEM_SHARED((NP, W), jnp.float32),
          pltpu.VMEM((NBATCH, BATCH), jnp.int32),
          pltpu.VMEM((NBATCH, BATCH), jnp.int32),
          pltpu.VMEM((BATCH, W), jnp.float32),
          pltpu.VMEM((BATCH, W), jnp.float32),
          pltpu.SemaphoreType.DMA,
          pltpu.SemaphoreType.DMA,
      ],
  )
  def segsum(h2d, src, dst, zeros, out, agg, srcv, dstv, rows0, rows1,
             sem0, sem1):
    cid = lax.axis_index("c")
    sid = lax.axis_index("s")
    wid = cid * NS + sid

    pltpu.sync_copy(dst.at[wid], dstv)

    for c in range(C):
      pltpu.sync_copy(src.at[c * NW + wid], srcv)
      pltpu.sync_copy(zeros, agg.at[pl.ds(sid * RPT, RPT)])
      plsc.subcore_barrier()

      pltpu.make_async_copy(h2d.at[srcv.at[0]], rows0, sem0).start()

      def pair(jj, carry):
        j0 = jj * 2
        j1 = j0 + 1
        pltpu.make_async_copy(h2d.at[srcv.at[j0]], rows0, sem0).wait()
        pltpu.make_async_copy(h2d.at[srcv.at[j1]], rows1, sem1).start()
        pltpu.sync_copy(rows0, agg.at[dstv.at[j0]], add=True)
        pltpu.make_async_copy(h2d.at[srcv.at[j1]], rows1, sem1).wait()

        @pl.when(jj + 1 < npair)
        def _():
          pltpu.make_async_copy(h2d.at[srcv.at[j0 + 2]], rows0, sem0).start()

        pltpu.sync_copy(rows1, agg.at[dstv.at[j1]], add=True)
        return carry

      lax.fori_loop(0, npair, pair, 0)

      plsc.subcore_barrier()
      pltpu.sync_copy(
          agg.at[pl.ds(sid * RPT, RPT)],
          out.at[cid * C + c, pl.ds(sid * RPT, RPT)])

  return segsum


NBH = 40
EPT = E // NS


def _make_segsum_cc():
  mesh = plsc.VectorSubcoreMesh(
      core_axis_name="c", subcore_axis_name="s", num_cores=NC,
      num_subcores=NS)
  npair = NBH // 2

  @functools.partial(
      pl.kernel,
      out_type=jax.ShapeDtypeStruct((4, NP, 128), jnp.float32),
      mesh=mesh,
      scratch_types=[
          pltpu.VMEM_SHARED((NP, 128), jnp.float32),
          pltpu.VMEM((NBH, BATCH), jnp.int32),
          pltpu.VMEM((NBH, BATCH), jnp.int32),
          pltpu.VMEM((BATCH, 128), jnp.float32),
          pltpu.VMEM((BATCH, 128), jnp.float32),
          pltpu.SemaphoreType.DMA,
          pltpu.SemaphoreType.DMA,
      ],
  )
  def segsum(h2d, src, dst, zeros, out, agg, srcv, dstv, rows0, rows1,
             sem0, sem1):
    cid = lax.axis_index("c")
    sid = lax.axis_index("s")

    def chunk(k, carry):
      c = cid * 2 + k
      pltpu.sync_copy(zeros, agg.at[pl.ds(sid * RPT, RPT)])
      plsc.subcore_barrier()

      def half(m, carry2):
        pltpu.sync_copy(src.at[(c * 2 + m) * NS + sid], srcv)
        pltpu.sync_copy(dst.at[m * NS + sid], dstv)
        pltpu.make_async_copy(h2d.at[srcv.at[0]], rows0, sem0).start()

        def pair(jj, carry3):
          j0 = jj * 2
          j1 = j0 + 1
          pltpu.make_async_copy(h2d.at[srcv.at[j0]], rows0, sem0).wait()
          pltpu.make_async_copy(h2d.at[srcv.at[j1]], rows1, sem1).start()
          pltpu.sync_copy(rows0, agg.at[dstv.at[j0]], add=True)
          pltpu.make_async_copy(h2d.at[srcv.at[j1]], rows1, sem1).wait()

          @pl.when(jj + 1 < npair)
          def _():
            pltpu.make_async_copy(h2d.at[srcv.at[j0 + 2]], rows0,
                                  sem0).start()

          pltpu.sync_copy(rows1, agg.at[dstv.at[j1]], add=True)
          return carry3

        lax.fori_loop(0, npair, pair, 0)
        return carry2

      lax.fori_loop(0, 2, half, 0)

      plsc.subcore_barrier()
      pltpu.sync_copy(
          agg.at[pl.ds(sid * RPT, RPT)],
          out.at[c, pl.ds(sid * RPT, RPT)])
      return carry

    lax.fori_loop(0, 2, chunk, 0)

  return segsum


_RB = 2000
_TRB = 2000


def _mm(xs, w, c_out):
  ks = [x.shape[1] for x in xs]
  offs = [sum(ks[:i]) for i in range(len(ks))]

  def body(*refs):
    xrefs, w_ref, o_ref = refs[:-2], refs[-2], refs[-1]
    acc = None
    for x_ref, k, off in zip(xrefs, ks, offs):
      part = lax.dot_general(
          x_ref[...], w_ref[off:off + k, :], (((1,), (0,)), ((), ())),
          precision=lax.Precision.DEFAULT,
          preferred_element_type=jnp.float32)
      acc = part if acc is None else acc + part
    for co in range(c_out):
      o_ref[co] = acc[:, co * 128:(co + 1) * 128]

  return pl.pallas_call(
      body,
      grid=(N // _RB,),
      in_specs=[pl.BlockSpec((_RB, k), lambda r: (r, 0)) for k in ks]
      + [pl.BlockSpec(w.shape, lambda r: (0, 0))],
      out_specs=pl.BlockSpec((c_out, _RB, 128), lambda r: (0, r, 0)),
      out_shape=jax.ShapeDtypeStruct((c_out, N, 128), jnp.float32),
  )(*xs, w)


def _tanh_sum_body(a_ref, b_ref, o_ref):
  o_ref[...] = jnp.tanh(a_ref[0] + b_ref[0]).astype(o_ref.dtype)


def _tanh_sum(p, C, dtype=jnp.float32):
  return pl.pallas_call(
      _tanh_sum_body,
      grid=(C, N // _TRB),
      in_specs=[
          pl.BlockSpec((1, _TRB, 128), lambda c, r: (c, r, 0)),
          pl.BlockSpec((1, _TRB, 128), lambda c, r: (C + c, r, 0)),
      ],
      out_specs=pl.BlockSpec((_TRB, 128), lambda c, r: (r, c)),
      out_shape=jax.ShapeDtypeStruct((N, C * 128), dtype),
  )(p, p)


def _tanh_mm(p, xs, w, c_out, hacc=None):
  ks = [x.shape[1] for x in xs]
  offs = [sum(ks[:i]) for i in range(len(ks))]
  base = sum(ks)
  n_acc = 0 if hacc is None else 1

  def body(*refs):
    p_ref = refs[0]
    xrefs = refs[1:1 + len(xs)]
    w_ref = refs[1 + len(xs)]
    hacc_ref = refs[2 + len(xs)] if n_acc else None
    x_out, h_out = refs[2 + len(xs) + n_acc:]
    xnew = jnp.concatenate([jnp.tanh(p_ref[c]) for c in range(4)],
                           axis=1).astype(jnp.bfloat16)
    x_out[...] = xnew
    acc = lax.dot_general(
        xnew, w_ref[base:base + 512, :], (((1,), (0,)), ((), ())),
        precision=lax.Precision.DEFAULT,
        preferred_element_type=jnp.float32)
    for x_ref, k, off in zip(xrefs, ks, offs):
      acc += lax.dot_general(
          x_ref[...], w_ref[off:off + k, :], (((1,), (0,)), ((), ())),
          precision=lax.Precision.DEFAULT,
          preferred_element_type=jnp.float32)
    for co in range(c_out):
      part = acc[:, co * 128:(co + 1) * 128]
      if n_acc:
        part = part + hacc_ref[co]
      h_out[co] = part

  extra_in = [] if hacc is None else [hacc]
  return pl.pallas_call(
      body,
      grid=(N // _RB,),
      in_specs=[pl.BlockSpec((4, _RB, 128), lambda r: (0, r, 0))]
      + [pl.BlockSpec((_RB, k), lambda r: (r, 0)) for k in ks]
      + [pl.BlockSpec(w.shape, lambda r: (0, 0))]
      + [pl.BlockSpec((c_out, _RB, 128), lambda r: (0, r, 0))
         for _ in extra_in],
      out_specs=[
          pl.BlockSpec((_RB, 512), lambda r: (r, 0)),
          pl.BlockSpec((c_out, _RB, 128), lambda r: (0, r, 0)),
      ],
      out_shape=[
          jax.ShapeDtypeStruct((N, 512), jnp.bfloat16),
          jax.ShapeDtypeStruct((c_out, N, 128), jnp.float32),
      ],
  )(p, *xs, w, *extra_in)


def kernel(features, edge_index, W0, W1, W2, W3):
  src = edge_index[0].astype(jnp.int32)
  dst = edge_index[1].astype(jnp.int32)

  npad = EPW_PAD - EPW
  pad_src = jnp.broadcast_to((jnp.arange(npad, dtype=jnp.int32) * 83) % N,
                             (NW, npad))
  pad_dst = jnp.broadcast_to(
      N + (jnp.arange(npad, dtype=jnp.int32) % (NP - N)), (NW, npad))
  src_w = jnp.concatenate([src.reshape(NW, EPW), pad_src], axis=1)
  dst_w = jnp.concatenate([dst.reshape(NW, EPW), pad_dst], axis=1)
  dst_t = dst_w.reshape(NW, NBATCH, BATCH)
  src1 = src_w.reshape(NW, NBATCH, BATCH)

  nptc = 2 * NBH * BATCH - EPT
  pad_src_t = jnp.broadcast_to((jnp.arange(nptc, dtype=jnp.int32) * 83) % N,
                               (NS, nptc))
  pad_dst_t = jnp.broadcast_to(
      N + (jnp.arange(nptc, dtype=jnp.int32) % (NP - N)), (NS, nptc))
  src_tile = jnp.concatenate([src.reshape(NS, EPT), pad_src_t], axis=1)
  dst_tile = jnp.concatenate([dst.reshape(NS, EPT), pad_dst_t], axis=1)
  dst_cc = dst_tile.reshape(NS, 2, NBH, BATCH).transpose(1, 0, 2, 3)
  dst_cc = dst_cc.reshape(2 * NS, NBH, BATCH)
  src_cc = (jnp.arange(4, dtype=jnp.int32)[:, None, None] * N
            + src_tile[None]).reshape(4, NS, 2, NBH, BATCH)
  src_cc = src_cc.transpose(0, 2, 1, 3, 4).reshape(8 * NS, NBH, BATCH)

  zeros128 = jnp.zeros((RPT, 128), jnp.float32)
  seg_cc = _make_segsum_cc()
  seg1 = _make_segsum(1, 128)

  featb = features.astype(jnp.bfloat16)
  W0b = W0.astype(jnp.bfloat16)
  W1b = W1.astype(jnp.bfloat16)
  W2b = W2.astype(jnp.bfloat16)

  h0 = _mm([featb], W0b, 4)
  p0 = seg_cc(h0.reshape(4 * N, 128), src_cc, dst_cc, zeros128)

  x0, h1 = _tanh_mm(p0, [], W1b, 4)
  p1 = seg_cc(h1.reshape(4 * N, 128), src_cc, dst_cc, zeros128)

  x1, h2 = _tanh_mm(p1, [x0], W2b, 4)
  p2 = seg_cc(h2.reshape(4 * N, 128), src_cc, dst_cc, zeros128)

  W3p = jnp.pad(W3, ((0, 0), (0, 64))).astype(jnp.bfloat16)
  h3a = _mm([x0, x1], W3p[:1024], 1)
  _, h3 = _tanh_mm(p2, [], W3p[1024:], 1, hacc=h3a)
  p3 = seg1(h3.reshape(N, 128), src1, dst_t, zeros128)
  out = _tanh_sum(p3, 1)
  return out[:, :64]
